# Initial kernel scaffold; baseline (speedup 1.0000x reference)
import jax
import jax.numpy as jnp
from jax.experimental import pallas as pl


def kernel(x, codebook):
    raise NotImplementedError("write your pallas kernel here")


# speedup vs baseline: 2.1836x; 2.1836x over previous
import functools

import jax
import jax.numpy as jnp
from jax import lax
from jax.experimental import pallas as pl
from jax.experimental.pallas import tpu as pltpu
from jax.experimental.pallas import tpu_sc as plsc

_CODEBOOK_SIZE = 8192
_DIM = 32
_ROW_BLOCK = 256


def _vq_tc_kernel(x_ref, cb_ref, idx_ref, oh_ref):
    xb = x_ref[...]
    cb = cb_ref[...]
    a2 = jnp.sum(xb * xb, axis=1, keepdims=True)
    b2 = jnp.sum(cb * cb, axis=1)[None, :]
    ab = lax.dot_general(
        xb, cb, (((1,), (1,)), ((), ())),
        precision=lax.Precision.HIGHEST,
        preferred_element_type=jnp.float32)
    dist = a2 - 2.0 * ab + b2
    idx = jnp.argmin(dist, axis=1).astype(jnp.int32)
    iota = lax.broadcasted_iota(jnp.int32, (_ROW_BLOCK, _CODEBOOK_SIZE), 1)
    oh_ref[...] = (idx[:, None] == iota).astype(jnp.float32)
    idx_ref[0, 0, :] = idx


def _argmin_onehot(flat, cb):
    n = flat.shape[0]
    nblk = n // _ROW_BLOCK
    idx3, onehot = pl.pallas_call(
        _vq_tc_kernel,
        grid=(nblk,),
        in_specs=[
            pl.BlockSpec((_ROW_BLOCK, _DIM), lambda i: (i, 0)),
            pl.BlockSpec((_CODEBOOK_SIZE, _DIM), lambda i: (0, 0)),
        ],
        out_specs=[
            pl.BlockSpec((1, 1, _ROW_BLOCK), lambda i: (i, 0, 0)),
            pl.BlockSpec((_ROW_BLOCK, _CODEBOOK_SIZE), lambda i: (i, 0)),
        ],
        out_shape=[
            jax.ShapeDtypeStruct((nblk, 1, _ROW_BLOCK), jnp.int32),
            jax.ShapeDtypeStruct((n, _CODEBOOK_SIZE), jnp.float32),
        ],
        compiler_params=pltpu.CompilerParams(
            dimension_semantics=("arbitrary",)),
    )(flat, cb)
    return idx3.reshape(n), onehot


def _make_sc_gather(n):
    num_workers = 32
    b_per_w = n // num_workers
    mesh = plsc.VectorSubcoreMesh(core_axis_name="c", subcore_axis_name="s")

    @functools.partial(
        pl.kernel,
        out_type=jax.ShapeDtypeStruct((n, _DIM), jnp.float32),
        mesh=mesh,
        scratch_types=[
            pltpu.VMEM((b_per_w,), jnp.int32),
            pltpu.VMEM((b_per_w, _DIM), jnp.float32),
            pltpu.SemaphoreType.DMA,
        ],
        compiler_params=pltpu.CompilerParams(use_tc_tiling_on_sc=False),
    )
    def gather(cb_hbm, idx_hbm, out_hbm, idx_v, rows_v, sem):
        wid = lax.axis_index("s") * 2 + lax.axis_index("c")
        base = wid * b_per_w
        pltpu.sync_copy(idx_hbm.at[pl.ds(base, b_per_w)], idx_v)
        pltpu.async_copy(cb_hbm.at[idx_v], rows_v, sem).wait()
        pltpu.sync_copy(rows_v, out_hbm.at[pl.ds(base, b_per_w)])

    return gather


def kernel(x, codebook):
    cb = jnp.asarray(codebook, dtype=jnp.float32)
    flat = jnp.reshape(x, (-1, _DIM))
    n = flat.shape[0]
    idx_flat, onehot = _argmin_onehot(flat, cb)
    quantized = _make_sc_gather(n)(cb, idx_flat)
    return (
        jnp.reshape(quantized, x.shape),
        jnp.reshape(onehot, x.shape[:-1] + (_CODEBOOK_SIZE,)),
        jnp.reshape(idx_flat, x.shape[:-1]),
        x,
    )


# speedup vs baseline: 2.1888x; 1.0024x over previous
import functools

import jax
import jax.numpy as jnp
from jax import lax
from jax.experimental import pallas as pl
from jax.experimental.pallas import tpu as pltpu
from jax.experimental.pallas import tpu_sc as plsc

_CODEBOOK_SIZE = 8192
_DIM = 32
_ROW_BLOCK = 256


def _vq_tc_kernel(x_ref, cb_ref, idx_ref, oh_ref, b2_ref):
    @pl.when(pl.program_id(0) == 0)
    def _():
        cbw = cb_ref[...]
        b2_ref[...] = jnp.sum(cbw * cbw, axis=1)[None, :]

    xb = x_ref[...]
    cb = cb_ref[...]
    a2 = jnp.sum(xb * xb, axis=1, keepdims=True)
    ab = lax.dot_general(
        xb, cb, (((1,), (1,)), ((), ())),
        precision=lax.Precision.HIGHEST,
        preferred_element_type=jnp.float32)
    dist = a2 - 2.0 * ab + b2_ref[...]
    idx = jnp.argmin(dist, axis=1).astype(jnp.int32)
    iota = lax.broadcasted_iota(jnp.int32, (_ROW_BLOCK, _CODEBOOK_SIZE), 1)
    oh_ref[...] = (idx[:, None] == iota).astype(jnp.float32)
    idx_ref[0, 0, :] = idx


def _argmin_onehot(flat, cb):
    n = flat.shape[0]
    nblk = n // _ROW_BLOCK
    idx3, onehot = pl.pallas_call(
        _vq_tc_kernel,
        grid=(nblk,),
        in_specs=[
            pl.BlockSpec((_ROW_BLOCK, _DIM), lambda i: (i, 0)),
            pl.BlockSpec((_CODEBOOK_SIZE, _DIM), lambda i: (0, 0)),
        ],
        out_specs=[
            pl.BlockSpec((1, 1, _ROW_BLOCK), lambda i: (i, 0, 0)),
            pl.BlockSpec((_ROW_BLOCK, _CODEBOOK_SIZE), lambda i: (i, 0)),
        ],
        out_shape=[
            jax.ShapeDtypeStruct((nblk, 1, _ROW_BLOCK), jnp.int32),
            jax.ShapeDtypeStruct((n, _CODEBOOK_SIZE), jnp.float32),
        ],
        scratch_shapes=[pltpu.VMEM((1, _CODEBOOK_SIZE), jnp.float32)],
        compiler_params=pltpu.CompilerParams(
            dimension_semantics=("arbitrary",)),
    )(flat, cb)
    return idx3.reshape(n), onehot


def _make_sc_gather(n):
    num_workers = 32
    b_per_w = n // num_workers
    mesh = plsc.VectorSubcoreMesh(core_axis_name="c", subcore_axis_name="s")

    @functools.partial(
        pl.kernel,
        out_type=jax.ShapeDtypeStruct((n, _DIM), jnp.float32),
        mesh=mesh,
        scratch_types=[
            pltpu.VMEM((b_per_w,), jnp.int32),
            pltpu.VMEM((b_per_w, _DIM), jnp.float32),
            pltpu.SemaphoreType.DMA,
        ],
        compiler_params=pltpu.CompilerParams(use_tc_tiling_on_sc=False),
    )
    def gather(cb_hbm, idx_hbm, out_hbm, idx_v, rows_v, sem):
        wid = lax.axis_index("s") * 2 + lax.axis_index("c")
        base = wid * b_per_w
        pltpu.sync_copy(idx_hbm.at[pl.ds(base, b_per_w)], idx_v)
        pltpu.async_copy(cb_hbm.at[idx_v], rows_v, sem).wait()
        pltpu.sync_copy(rows_v, out_hbm.at[pl.ds(base, b_per_w)])

    return gather


def kernel(x, codebook):
    cb = jnp.asarray(codebook, dtype=jnp.float32)
    flat = jnp.reshape(x, (-1, _DIM))
    n = flat.shape[0]
    idx_flat, onehot = _argmin_onehot(flat, cb)
    quantized = _make_sc_gather(n)(cb, idx_flat)
    return (
        jnp.reshape(quantized, x.shape),
        jnp.reshape(onehot, x.shape[:-1] + (_CODEBOOK_SIZE,)),
        jnp.reshape(idx_flat, x.shape[:-1]),
        x,
    )


# speedup vs baseline: 2.3409x; 1.0695x over previous
import functools

import jax
import jax.numpy as jnp
from jax import lax
from jax.experimental import pallas as pl
from jax.experimental.pallas import tpu as pltpu
from jax.experimental.pallas import tpu_sc as plsc

_CODEBOOK_SIZE = 8192
_DIM = 32
_ROW_BLOCK = 256


def _vq_tc_kernel(x_ref, cbt_ref, idx_ref, oh_ref, b2_ref):
    @pl.when(pl.program_id(0) == 0)
    def _():
        cbw = cbt_ref[...]
        b2_ref[...] = jnp.sum(cbw * cbw, axis=0, keepdims=True)

    xb = x_ref[...]
    cbt = cbt_ref[...]
    a2 = jnp.sum(xb * xb, axis=1, keepdims=True)
    nab2 = lax.dot_general(
        xb * -2.0, cbt, (((1,), (0,)), ((), ())),
        precision=lax.Precision.HIGHEST,
        preferred_element_type=jnp.float32)
    dist = a2 + nab2 + b2_ref[...]
    idx = jnp.argmin(dist, axis=1).astype(jnp.int32)
    iota = lax.broadcasted_iota(jnp.int32, (_ROW_BLOCK, _CODEBOOK_SIZE), 1)
    oh_ref[...] = (idx[:, None] == iota).astype(jnp.float32)
    idx_ref[0, 0, :] = idx


def _argmin_onehot(flat, cbt):
    n = flat.shape[0]
    nblk = n // _ROW_BLOCK
    idx3, onehot = pl.pallas_call(
        _vq_tc_kernel,
        grid=(nblk,),
        in_specs=[
            pl.BlockSpec((_ROW_BLOCK, _DIM), lambda i: (i, 0)),
            pl.BlockSpec((_DIM, _CODEBOOK_SIZE), lambda i: (0, 0)),
        ],
        out_specs=[
            pl.BlockSpec((1, 1, _ROW_BLOCK), lambda i: (i, 0, 0)),
            pl.BlockSpec((_ROW_BLOCK, _CODEBOOK_SIZE), lambda i: (i, 0)),
        ],
        out_shape=[
            jax.ShapeDtypeStruct((nblk, 1, _ROW_BLOCK), jnp.int32),
            jax.ShapeDtypeStruct((n, _CODEBOOK_SIZE), jnp.float32),
        ],
        scratch_shapes=[pltpu.VMEM((1, _CODEBOOK_SIZE), jnp.float32)],
        compiler_params=pltpu.CompilerParams(
            dimension_semantics=("arbitrary",)),
    )(flat, cbt)
    return idx3.reshape(n), onehot


def _make_sc_gather(n):
    num_workers = 32
    b_per_w = n // num_workers
    mesh = plsc.VectorSubcoreMesh(core_axis_name="c", subcore_axis_name="s")

    @functools.partial(
        pl.kernel,
        out_type=jax.ShapeDtypeStruct((n, _DIM), jnp.float32),
        mesh=mesh,
        scratch_types=[
            pltpu.VMEM((b_per_w,), jnp.int32),
            pltpu.VMEM((b_per_w, _DIM), jnp.float32),
            pltpu.SemaphoreType.DMA,
        ],
        compiler_params=pltpu.CompilerParams(use_tc_tiling_on_sc=False),
    )
    def gather(cb_hbm, idx_hbm, out_hbm, idx_v, rows_v, sem):
        wid = lax.axis_index("s") * 2 + lax.axis_index("c")
        base = wid * b_per_w
        pltpu.sync_copy(idx_hbm.at[pl.ds(base, b_per_w)], idx_v)
        pltpu.async_copy(cb_hbm.at[idx_v], rows_v, sem).wait()
        pltpu.sync_copy(rows_v, out_hbm.at[pl.ds(base, b_per_w)])

    return gather


def kernel(x, codebook):
    cb = jnp.asarray(codebook, dtype=jnp.float32)
    flat = jnp.reshape(x, (-1, _DIM))
    n = flat.shape[0]
    idx_flat, onehot = _argmin_onehot(flat, cb.T)
    quantized = _make_sc_gather(n)(cb, idx_flat)
    return (
        jnp.reshape(quantized, x.shape),
        jnp.reshape(onehot, x.shape[:-1] + (_CODEBOOK_SIZE,)),
        jnp.reshape(idx_flat, x.shape[:-1]),
        x,
    )


# speedup vs baseline: 2.3718x; 1.0132x over previous
import functools

import jax
import jax.numpy as jnp
from jax import lax
from jax.experimental import pallas as pl
from jax.experimental.pallas import tpu as pltpu
from jax.experimental.pallas import tpu_sc as plsc

_CODEBOOK_SIZE = 8192
_DIM = 32
_ROW_BLOCK = 256


def _vq_tc_kernel(x_ref, cbt_ref, idx_ref, oh_ref, b2_ref, idxs_ref):
    @pl.when(pl.program_id(0) == 0)
    def _():
        cbw = cbt_ref[...]
        b2_ref[...] = jnp.sum(cbw * cbw, axis=0, keepdims=True)

    prev_idx = idxs_ref[0, :]
    hi = lax.shift_right_logical(prev_idx, 7)
    lo = lax.bitwise_and(prev_idx, 127)
    iota_hi = lax.broadcasted_iota(jnp.int32, (_ROW_BLOCK, 64), 1)
    iota_lo = lax.broadcasted_iota(jnp.int32, (_ROW_BLOCK, 128), 1)
    u = (hi[:, None] == iota_hi).astype(jnp.float32)
    v = (lo[:, None] == iota_lo).astype(jnp.float32)
    oh_ref[...] = (u[:, :, None] * v[:, None, :]).reshape(
        _ROW_BLOCK, _CODEBOOK_SIZE)

    xb = x_ref[...]
    cbt = cbt_ref[...]
    a2 = jnp.sum(xb * xb, axis=1, keepdims=True)
    nab2 = lax.dot_general(
        xb * -2.0, cbt, (((1,), (0,)), ((), ())),
        precision=lax.Precision.HIGHEST,
        preferred_element_type=jnp.float32)
    dist = a2 + nab2 + b2_ref[...]
    idx = jnp.argmin(dist, axis=1).astype(jnp.int32)
    idxs_ref[0, :] = idx
    idx_ref[0, 0, :] = idx


def _argmin_onehot(flat, cbt):
    n = flat.shape[0]
    nblk = n // _ROW_BLOCK
    idx3, onehot = pl.pallas_call(
        _vq_tc_kernel,
        grid=(nblk + 1,),
        in_specs=[
            pl.BlockSpec(
                (_ROW_BLOCK, _DIM),
                lambda i: (jnp.minimum(i, nblk - 1), 0)),
            pl.BlockSpec((_DIM, _CODEBOOK_SIZE), lambda i: (0, 0)),
        ],
        out_specs=[
            pl.BlockSpec(
                (1, 1, _ROW_BLOCK),
                lambda i: (jnp.minimum(i, nblk - 1), 0, 0)),
            pl.BlockSpec(
                (_ROW_BLOCK, _CODEBOOK_SIZE),
                lambda i: (jnp.maximum(i - 1, 0), 0)),
        ],
        out_shape=[
            jax.ShapeDtypeStruct((nblk, 1, _ROW_BLOCK), jnp.int32),
            jax.ShapeDtypeStruct((n, _CODEBOOK_SIZE), jnp.float32),
        ],
        scratch_shapes=[
            pltpu.VMEM((1, _CODEBOOK_SIZE), jnp.float32),
            pltpu.VMEM((1, _ROW_BLOCK), jnp.int32),
        ],
        compiler_params=pltpu.CompilerParams(
            dimension_semantics=("arbitrary",)),
    )(flat, cbt)
    return idx3.reshape(n), onehot


def _make_sc_gather(n):
    num_workers = 32
    b_per_w = n // num_workers
    mesh = plsc.VectorSubcoreMesh(core_axis_name="c", subcore_axis_name="s")

    @functools.partial(
        pl.kernel,
        out_type=jax.ShapeDtypeStruct((n, _DIM), jnp.float32),
        mesh=mesh,
        scratch_types=[
            pltpu.VMEM((b_per_w,), jnp.int32),
            pltpu.VMEM((b_per_w, _DIM), jnp.float32),
            pltpu.SemaphoreType.DMA,
        ],
        compiler_params=pltpu.CompilerParams(use_tc_tiling_on_sc=False),
    )
    def gather(cb_hbm, idx_hbm, out_hbm, idx_v, rows_v, sem):
        wid = lax.axis_index("s") * 2 + lax.axis_index("c")
        base = wid * b_per_w
        pltpu.sync_copy(idx_hbm.at[pl.ds(base, b_per_w)], idx_v)
        pltpu.async_copy(cb_hbm.at[idx_v], rows_v, sem).wait()
        pltpu.sync_copy(rows_v, out_hbm.at[pl.ds(base, b_per_w)])

    return gather


def kernel(x, codebook):
    cb = jnp.asarray(codebook, dtype=jnp.float32)
    flat = jnp.reshape(x, (-1, _DIM))
    n = flat.shape[0]
    idx_flat, onehot = _argmin_onehot(flat, cb.T)
    quantized = _make_sc_gather(n)(cb, idx_flat)
    return (
        jnp.reshape(quantized, x.shape),
        jnp.reshape(onehot, x.shape[:-1] + (_CODEBOOK_SIZE,)),
        jnp.reshape(idx_flat, x.shape[:-1]),
        x,
    )
